# baseline (speedup 1.0000x reference)
import functools

import jax
import jax.numpy as jnp
from jax import lax
from jax.experimental import pallas as pl
from jax.experimental.pallas import tpu as pltpu
from jax.experimental.pallas import tpu_sc as plsc

_INFO = plsc.get_sparse_core_info()
_NC = _INFO.num_cores
_NS = _INFO.num_subcores
_NW = _NC * _NS

_CHUNK = 128
_NBUF = 5


@functools.partial(jax.jit, static_argnames=())
def _gather_rows(idx_grouped, table):
    nw, nchunk, chunk = idx_grouped.shape
    n_rows = nw * nchunk * chunk
    d = table.shape[1]
    nbuf = _NBUF
    assert nchunk % nbuf == 0

    mesh = plsc.VectorSubcoreMesh(core_axis_name="c", subcore_axis_name="s")

    @functools.partial(
        pl.kernel,
        mesh=mesh,
        out_type=jax.ShapeDtypeStruct((n_rows, d), jnp.float32),
        scratch_types=(
            [pltpu.VMEM((nchunk, chunk), jnp.int32)]
            + [pltpu.VMEM((chunk, d), jnp.float32)] * nbuf
            + [pltpu.SemaphoreType.DMA] * nbuf
        ),
    )
    def k(idx_hbm, table_hbm, out_hbm, idx_v, *bufs_and_sems):
        rows = bufs_and_sems[:nbuf]
        gsem = bufs_and_sems[nbuf:2 * nbuf]
        sid = lax.axis_index("s")
        cid = lax.axis_index("c")

        @pl.when(sid < _NS // 2)
        def _():
            def span(h):
                wid = (2 * sid + h) * _NC + cid
                base = wid * (nchunk * chunk)
                pltpu.sync_copy(idx_hbm.at[wid], idx_v)

                def gcopy(c, u):
                    return pltpu.make_async_copy(
                        table_hbm.at[idx_v.at[c]], rows[u], gsem[u])

                for u in range(nbuf):
                    gcopy(u, u).start()

                def body(i, carry):
                    cb = i * nbuf
                    for u in range(nbuf):
                        c = cb + u
                        gcopy(c, u).wait()

                        @pl.when(c + nbuf < nchunk)
                        def _():
                            gcopy(c + nbuf, u).start()

                    return carry

                lax.fori_loop(0, nchunk // nbuf, body, 0)
                pltpu.sync_copy(rows[0], out_hbm.at[pl.ds(base, chunk)])

            span(0)
            span(1)

    return k(idx_grouped, table)


def kernel(indices, table):
    b, l = indices.shape
    d = table.shape[1]
    n = b * l
    rows_per_w = n // _NW
    nchunk = rows_per_w // _CHUNK

    idx_t = jnp.transpose(indices.astype(jnp.int32), (1, 0))
    idx_grouped = idx_t.reshape(_NW, nchunk, _CHUNK)
    out_flat = _gather_rows(idx_grouped, table)
    return out_flat.reshape(l, b, d)


# speedup vs baseline: 1.0705x; 1.0705x over previous
import functools

import jax
import jax.numpy as jnp
from jax import lax
from jax.experimental import pallas as pl
from jax.experimental.pallas import tpu as pltpu
from jax.experimental.pallas import tpu_sc as plsc

_INFO = plsc.get_sparse_core_info()
_NC = _INFO.num_cores
_NS = _INFO.num_subcores
_NW = _NC * _NS

_CHUNK = 128


@functools.partial(jax.jit, static_argnames=())
def _gather_rows(idx_grouped, table):
    nw, nchunk, chunk = idx_grouped.shape
    n_rows = nw * nchunk * chunk
    d = table.shape[1]
    npair = nchunk // 2
    assert nchunk % 2 == 0

    mesh = plsc.VectorSubcoreMesh(core_axis_name="c", subcore_axis_name="s")

    @functools.partial(
        pl.kernel,
        mesh=mesh,
        out_type=jax.ShapeDtypeStruct((n_rows, d), jnp.float32),
        scratch_types=(
            [pltpu.VMEM((nchunk, chunk), jnp.int32)]
            + [pltpu.VMEM((chunk, d), jnp.float32)] * 4
            + [pltpu.MemorySpace.VMEM_SHARED((_NS, 2, chunk, d), jnp.float32)]
            + [pltpu.SemaphoreType.DMA] * 10
        ),
    )
    def k(idx_hbm, table_hbm, out_hbm, idx_v, b0, b1, b2, b3, shared, *sems):
        rows = (b0, b1, b2, b3)
        gsem = sems[0:4]
        ssem = sems[4:6]
        psem = sems[6:8]
        dsem = sems[8:10]
        sid = lax.axis_index("s")
        wid = sid * _NC + lax.axis_index("c")
        base = wid * (nchunk * chunk)
        pltpu.sync_copy(idx_hbm.at[wid], idx_v)

        def gcopy(c, u):
            return pltpu.make_async_copy(
                table_hbm.at[idx_v.at[c]], rows[u], gsem[u])

        def scopy(c, u, k_):
            return pltpu.make_async_copy(
                rows[u], out_hbm.at[pl.ds(base + c * chunk, chunk)], ssem[k_])

        def pcopy(u, sp):
            return pltpu.make_async_copy(rows[u], shared.at[sid, sp], psem[sp])

        def dcopy(c, sp):
            return pltpu.make_async_copy(
                shared.at[sid, sp],
                out_hbm.at[pl.ds(base + c * chunk, chunk)], dsem[sp])

        for c in range(4):
            gcopy(c, c).start()

        def body(i2, carry):
            for k_ in range(2):
                p = 2 * i2 + k_
                u0, u1 = (0, 1) if k_ == 0 else (2, 3)

                @pl.when(p < npair)
                def _():
                    c = 2 * p
                    @pl.when(p >= 2)
                    def _():
                        dcopy(2 * (p - 2) + 1, k_).wait()

                    gcopy(c, u0).wait()
                    scopy(c, u0, k_).start()
                    gcopy(c + 1, u1).wait()
                    pcopy(u1, k_).start()
                    pcopy(u1, k_).wait()
                    dcopy(c + 1, k_).start()

                    pg = p + 2
                    @pl.when(pg < npair)
                    def _():
                        scopy(c, u0, k_).wait()
                        gcopy(2 * pg, u0).start()
                        gcopy(2 * pg + 1, u1).start()

            return carry

        lax.fori_loop(0, (npair + 1) // 2, body, 0)
        scopy(2 * (npair - 1), 0, 0).wait()
        scopy(2 * (npair - 2), 2, 1).wait()
        dcopy(2 * (npair - 1) + 1, 0).wait()
        dcopy(2 * (npair - 2) + 1, 1).wait()

    return k(idx_grouped, table)


def kernel(indices, table):
    b, l = indices.shape
    d = table.shape[1]
    n = b * l
    rows_per_w = n // _NW
    nchunk = rows_per_w // _CHUNK
    assert rows_per_w % _CHUNK == 0 and n % _NW == 0

    idx_t = jnp.transpose(indices.astype(jnp.int32), (1, 0))
    idx_grouped = idx_t.reshape(_NW, nchunk, _CHUNK)
    out_flat = _gather_rows(idx_grouped, table)
    return out_flat.reshape(l, b, d)
